# Initial kernel scaffold; baseline (speedup 1.0000x reference)
import jax
import jax.numpy as jnp
from jax.experimental import pallas as pl


def kernel(x, edge_index, W_pn, b_pn, ln1_g, ln1_b, W_pk, b_pk, ln2_g, ln2_b, W_pe, b_pe, W_an, b_an, W_ih, b_ih, W_hh, b_hh, ln3_g, ln3_b, W_cp, b_cp, ln4_g, ln4_b):
    raise NotImplementedError("write your pallas kernel here")


# speedup vs baseline: 2.6664x; 2.6664x over previous
import functools

import jax
import jax.numpy as jnp
from jax import lax
from jax.experimental import pallas as pl
from jax.experimental.pallas import tpu as pltpu
from jax.experimental.pallas import tpu_sc as plsc

NN = 10000
NP = 10240
EE = 320000
DD = 128
KP = 20
NC = 2
NS = 16
NW = NC * NS
EW = EE // NW
CH = 80
NCH = EW // CH
GS = 25
NG = NCH // GS
SLICE = NP // NS
BN = 1000
BE = 512

_EPS = 1e-5


def _ln_lanes(y, g, b, n_lanes):
    mu = jnp.sum(y, axis=1, keepdims=True) / n_lanes
    diff = y - mu
    var = jnp.sum(diff * diff, axis=1, keepdims=True) / n_lanes
    return diff * lax.rsqrt(var + _EPS) * g + b


def _pre_body(x_ref, wpn_ref, bpn_ref, g1_ref, b1_ref, wan_ref, ban_ref,
              wped_ref, wpes_ref, bpe_ref, proj_ref, hv_ref, ld_ref, ls_ref):
    x = x_ref[...]
    p = jnp.dot(x, wpn_ref[...], preferred_element_type=jnp.float32) + bpn_ref[...]
    mask = lax.broadcasted_iota(jnp.int32, p.shape, 1) < KP
    mu = jnp.sum(p, axis=1, keepdims=True) / KP
    diff = jnp.where(mask, p - mu, 0.0)
    var = jnp.sum(diff * diff, axis=1, keepdims=True) / KP
    proj_ref[...] = jnp.maximum(diff * lax.rsqrt(var + _EPS) * g1_ref[...] + b1_ref[...], 0.0)
    hv_ref[...] = jnp.dot(x, wan_ref[...], preferred_element_type=jnp.float32) + ban_ref[...]
    ld_ref[...] = jnp.sum(x * wped_ref[...], axis=1, keepdims=True) + bpe_ref[...]
    ls_ref[...] = jnp.sum(x * wpes_ref[...], axis=1, keepdims=True)


def _kron_body(a_ref, b_ref, w_ref, bpk_ref, g2_ref, b2_ref, ke_ref):
    a = a_ref[...]
    b = b_ref[...]
    k3 = (a[:, :, None] * b[:, None, :]).reshape(BE, 1024)
    pre = jnp.dot(k3, w_ref[...], preferred_element_type=jnp.float32) + bpk_ref[...]
    ke_ref[...] = jnp.maximum(_ln_lanes(pre, g2_ref[...], b2_ref[...], DD), 0.0)


def _fin_body(x_ref, n0_ref, n1_ref, s0_ref, s1_ref, k0_ref, k1_ref,
              wih_ref, bih_ref, whh_ref, bhh_ref, g3_ref, b3_ref,
              wch_ref, wck_ref, bcp_ref, g4_ref, b4_ref, out_ref):
    x = x_ref[...]
    s = jnp.maximum(s0_ref[...] + s1_ref[...], 1e-12)
    ctx = jnp.maximum((n0_ref[...] + n1_ref[...]) / s, 0.0)
    nk = k0_ref[...] + k1_ref[...]
    gi = jnp.dot(ctx, wih_ref[...], preferred_element_type=jnp.float32) + bih_ref[...]
    gh = jnp.dot(x, whh_ref[...], preferred_element_type=jnp.float32) + bhh_ref[...]
    r = jax.nn.sigmoid(gi[:, :DD] + gh[:, :DD])
    z = jax.nn.sigmoid(gi[:, DD:2 * DD] + gh[:, DD:2 * DD])
    nc = jnp.tanh(gi[:, 2 * DD:] + r * gh[:, 2 * DD:])
    h = jnp.maximum((1.0 - z) * nc + z * x, 0.0)
    h = _ln_lanes(h, g3_ref[...], b3_ref[...], DD)
    o = (jnp.dot(h, wch_ref[...], preferred_element_type=jnp.float32)
         + jnp.dot(nk, wck_ref[...], preferred_element_type=jnp.float32) + bcp_ref[...])
    out_ref[...] = jnp.maximum(_ln_lanes(o, g4_ref[...], b4_ref[...], DD), 0.0)


def _edge_sc_body(proj_hbm, hv_hbm, ld_hbm, ls_hbm, srcw_hbm, dstw_hbm, zero_hbm,
                  a_hbm, b_hbm, num_hbm, s_hbm,
                  ld_v, ls_v, idxs_v, idxd_v, e_v, hs_v, a_v, b_v,
                  num_sh, s_sh):
    c = lax.axis_index("c")
    sid = lax.axis_index("s")
    wid = sid * NC + c
    base_e = wid * EW
    base_n = sid * SLICE

    pltpu.sync_copy(zero_hbm.at[pl.ds(base_n, SLICE)], num_sh.at[pl.ds(base_n, SLICE)])
    pltpu.sync_copy(zero_hbm.at[0, pl.ds(0, SLICE)], s_sh.at[pl.ds(base_n, SLICE)])

    pltpu.sync_copy(ld_hbm, ld_v)
    pltpu.sync_copy(ls_hbm, ls_v)
    plsc.subcore_barrier()

    def group(gq, carry):
        pltpu.sync_copy(srcw_hbm.at[wid, pl.ds(gq * GS, GS)], idxs_v)
        pltpu.sync_copy(dstw_hbm.at[wid, pl.ds(gq * GS, GS)], idxd_v)

        def chunk(jj, carry2):
            j = gq * GS + jj
            idx_s = idxs_v.at[jj]
            idx_d = idxd_v.at[jj]
            pltpu.sync_copy(proj_hbm.at[idx_s], a_v)
            pltpu.sync_copy(a_v, a_hbm.at[pl.ds(base_e + j * CH, CH)])
            pltpu.sync_copy(proj_hbm.at[idx_d], b_v)
            pltpu.sync_copy(b_v, b_hbm.at[pl.ds(base_e + j * CH, CH)])
            for g in range(CH // 16):
                ids = idxs_v[jj, pl.ds(g * 16, 16)]
                idd = idxd_v[jj, pl.ds(g * 16, 16)]
                lsg = plsc.load_gather(ls_v, [ids])
                ldg = plsc.load_gather(ld_v, [idd])
                e_v[pl.ds(g * 16, 16)] = jnp.exp(jnp.maximum(ldg + lsg, 0.0))
            pltpu.sync_copy(hv_hbm.at[idx_s], hs_v)

            def scale_row(r, carry3):
                w16 = plsc.load_gather(e_v, [jnp.full((16,), r, jnp.int32)])
                for q in range(DD // 16):
                    hs_v[r, pl.ds(q * 16, 16)] = hs_v[r, pl.ds(q * 16, 16)] * w16
                return carry3

            lax.fori_loop(0, CH, scale_row, 0, unroll=False)
            pltpu.sync_copy(hs_v, num_sh.at[idx_d], add=True)
            pltpu.sync_copy(e_v, s_sh.at[idx_d], add=True)
            return carry2

        lax.fori_loop(0, GS, chunk, 0, unroll=False)
        return carry

    lax.fori_loop(0, NG, group, 0, unroll=False)
    plsc.subcore_barrier()
    pltpu.sync_copy(num_sh.at[pl.ds(base_n, SLICE)], num_hbm.at[c, pl.ds(base_n, SLICE)])
    pltpu.sync_copy(s_sh.at[pl.ds(base_n, SLICE)], s_hbm.at[c, pl.ds(base_n, SLICE)])


def _nk_sc_body(ke_hbm, dstw_hbm, zero_hbm, nk_hbm, idxd_v, ke_v, nk_sh):
    c = lax.axis_index("c")
    sid = lax.axis_index("s")
    wid = sid * NC + c
    base_e = wid * EW
    base_n = sid * SLICE

    pltpu.sync_copy(zero_hbm.at[pl.ds(base_n, SLICE)], nk_sh.at[pl.ds(base_n, SLICE)])
    pltpu.sync_copy(dstw_hbm.at[wid], idxd_v)
    plsc.subcore_barrier()

    def chunk(j, carry):
        pltpu.sync_copy(ke_hbm.at[pl.ds(base_e + j * CH, CH)], ke_v)
        pltpu.sync_copy(ke_v, nk_sh.at[idxd_v.at[j]], add=True)
        return carry

    lax.fori_loop(0, NCH, chunk, 0, unroll=False)
    plsc.subcore_barrier()
    pltpu.sync_copy(nk_sh.at[pl.ds(base_n, SLICE)], nk_hbm.at[c, pl.ds(base_n, SLICE)])


_SC_MESH = plsc.VectorSubcoreMesh(core_axis_name="c", subcore_axis_name="s",
                                  num_cores=NC, num_subcores=NS)
_SC_PARAMS = pltpu.CompilerParams(use_tc_tiling_on_sc=False,
                                  needs_layout_passes=False)


def _make_edge_sc():
    f32 = jnp.float32
    return functools.partial(
        pl.kernel, mesh=_SC_MESH, compiler_params=_SC_PARAMS,
        out_type=(jax.ShapeDtypeStruct((EE, 32), f32),
                  jax.ShapeDtypeStruct((EE, 32), f32),
                  jax.ShapeDtypeStruct((NC, NP, DD), f32),
                  jax.ShapeDtypeStruct((NC, NP), f32)),
        scratch_types=[pltpu.VMEM((NN,), f32), pltpu.VMEM((NN,), f32),
                       pltpu.VMEM((GS, CH), jnp.int32), pltpu.VMEM((GS, CH), jnp.int32),
                       pltpu.VMEM((CH,), f32), pltpu.VMEM((CH, DD), f32),
                       pltpu.VMEM((CH, 32), f32), pltpu.VMEM((CH, 32), f32),
                       pltpu.VMEM_SHARED((NP, DD), f32), pltpu.VMEM_SHARED((NP,), f32)],
    )(_edge_sc_body)


def _make_nk_sc():
    f32 = jnp.float32
    return functools.partial(
        pl.kernel, mesh=_SC_MESH, compiler_params=_SC_PARAMS,
        out_type=jax.ShapeDtypeStruct((NC, NP, DD), f32),
        scratch_types=[pltpu.VMEM((NCH, CH), jnp.int32), pltpu.VMEM((CH, DD), f32),
                       pltpu.VMEM_SHARED((NP, DD), f32)],
    )(_nk_sc_body)


def kernel(x, edge_index, W_pn, b_pn, ln1_g, ln1_b, W_pk, b_pk, ln2_g, ln2_b,
           W_pe, b_pe, W_an, b_an, W_ih, b_ih, W_hh, b_hh, ln3_g, ln3_b,
           W_cp, b_cp, ln4_g, ln4_b):
    f32 = jnp.float32
    wpn32 = jnp.pad(W_pn, ((0, 0), (0, 12)))
    bpn32 = jnp.pad(b_pn, (0, 12)).reshape(1, 32)
    g1 = jnp.pad(ln1_g, (0, 12)).reshape(1, 32)
    b1 = jnp.pad(ln1_b, (0, 12)).reshape(1, 32)
    w1024 = jnp.pad(W_pk.reshape(KP, KP, DD), ((0, 12), (0, 12), (0, 0))).reshape(1024, DD)
    wped = W_pe[:DD].reshape(1, DD)
    wpes = W_pe[DD:].reshape(1, DD)
    bpe = b_pe.reshape(1, 1)
    srcw = edge_index[0].reshape(NW, NCH, CH)
    dstw = edge_index[1].reshape(NW, NCH, CH)
    zero_np = jnp.zeros((NP, DD), f32)

    full2 = lambda shp: pl.BlockSpec(shp, lambda i: (0, 0))

    proj, hv, ld, ls = pl.pallas_call(
        _pre_body,
        grid=(NN // BN,),
        in_specs=[pl.BlockSpec((BN, DD), lambda i: (i, 0)),
                  full2((DD, 32)), full2((1, 32)), full2((1, 32)), full2((1, 32)),
                  full2((DD, DD)), full2((1, DD)), full2((1, DD)), full2((1, DD)),
                  full2((1, 1))],
        out_specs=[pl.BlockSpec((BN, 32), lambda i: (i, 0)),
                   pl.BlockSpec((BN, DD), lambda i: (i, 0)),
                   pl.BlockSpec((BN, 1), lambda i: (i, 0)),
                   pl.BlockSpec((BN, 1), lambda i: (i, 0))],
        out_shape=[jax.ShapeDtypeStruct((NN, 32), f32),
                   jax.ShapeDtypeStruct((NN, DD), f32),
                   jax.ShapeDtypeStruct((NN, 1), f32),
                   jax.ShapeDtypeStruct((NN, 1), f32)],
    )(x, wpn32, bpn32, g1, b1, W_an, b_an.reshape(1, DD), wped, wpes, bpe)

    ld1 = ld.reshape(NN)
    ls1 = ls.reshape(NN)

    a_arr, b_arr, num_p, s_p = _make_edge_sc()(proj, hv, ld1, ls1, srcw, dstw, zero_np)

    ke = pl.pallas_call(
        _kron_body,
        grid=(EE // BE,),
        in_specs=[pl.BlockSpec((BE, 32), lambda i: (i, 0)),
                  pl.BlockSpec((BE, 32), lambda i: (i, 0)),
                  full2((1024, DD)), full2((1, DD)), full2((1, DD)), full2((1, DD))],
        out_specs=pl.BlockSpec((BE, DD), lambda i: (i, 0)),
        out_shape=jax.ShapeDtypeStruct((EE, DD), f32),
    )(a_arr, b_arr, w1024, b_pk.reshape(1, DD), ln2_g.reshape(1, DD), ln2_b.reshape(1, DD))

    nk_p = _make_nk_sc()(ke, dstw, zero_np)

    out = pl.pallas_call(
        _fin_body,
        grid=(NN // BN,),
        in_specs=[pl.BlockSpec((BN, DD), lambda i: (i, 0)),
                  pl.BlockSpec((BN, DD), lambda i: (i, 0)),
                  pl.BlockSpec((BN, DD), lambda i: (i, 0)),
                  pl.BlockSpec((BN, 1), lambda i: (i, 0)),
                  pl.BlockSpec((BN, 1), lambda i: (i, 0)),
                  pl.BlockSpec((BN, DD), lambda i: (i, 0)),
                  pl.BlockSpec((BN, DD), lambda i: (i, 0)),
                  full2((DD, 3 * DD)), full2((1, 3 * DD)),
                  full2((DD, 3 * DD)), full2((1, 3 * DD)),
                  full2((1, DD)), full2((1, DD)),
                  full2((DD, DD)), full2((DD, DD)), full2((1, DD)),
                  full2((1, DD)), full2((1, DD))],
        out_specs=pl.BlockSpec((BN, DD), lambda i: (i, 0)),
        out_shape=jax.ShapeDtypeStruct((NN, DD), f32),
    )(x, num_p[0, :NN], num_p[1, :NN], s_p[0, :NN].reshape(NN, 1), s_p[1, :NN].reshape(NN, 1),
      nk_p[0, :NN], nk_p[1, :NN],
      W_ih.T, b_ih.reshape(1, 3 * DD), W_hh.T, b_hh.reshape(1, 3 * DD),
      ln3_g.reshape(1, DD), ln3_b.reshape(1, DD),
      W_cp[:DD], W_cp[DD:], b_cp.reshape(1, DD),
      ln4_g.reshape(1, DD), ln4_b.reshape(1, DD))
    return out


# speedup vs baseline: 8.8129x; 3.3052x over previous
import functools

import numpy as _np

import jax
import jax.numpy as jnp
from jax import lax
from jax.experimental import pallas as pl
from jax.experimental.pallas import tpu as pltpu
from jax.experimental.pallas import tpu_sc as plsc

NN = 10000
NP = 10240
EE = 320000
DD = 128
KP = 20
NC = 2
NS = 16
NW = NC * NS
EW = EE // NW
CH = 80
NCH = EW // CH
GS = 25
NG = NCH // GS
SLICE = NP // NS
BN = 1000
BE = 2560

_EPS = 1e-5


def _ln_lanes(y, g, b, n_lanes):
    mu = jnp.sum(y, axis=1, keepdims=True) / n_lanes
    diff = y - mu
    var = jnp.sum(diff * diff, axis=1, keepdims=True) / n_lanes
    return diff * lax.rsqrt(var + _EPS) * g + b


def _pre_body(x_ref, wpn_ref, bpn_ref, g1_ref, b1_ref, wan_ref, ban_ref,
              wped_ref, wpes_ref, bpe_ref, proj_ref, hv_ref, ld_ref, ls_ref):
    x = x_ref[...]
    p = jnp.dot(x, wpn_ref[...], preferred_element_type=jnp.float32) + bpn_ref[...]
    mask = lax.broadcasted_iota(jnp.int32, p.shape, 1) < KP
    mu = jnp.sum(p, axis=1, keepdims=True) / KP
    diff = jnp.where(mask, p - mu, 0.0)
    var = jnp.sum(diff * diff, axis=1, keepdims=True) / KP
    proj_ref[...] = jnp.maximum(diff * lax.rsqrt(var + _EPS) * g1_ref[...] + b1_ref[...],
                                0.0).astype(jnp.bfloat16)
    hv_ref[...] = jnp.dot(x, wan_ref[...], preferred_element_type=jnp.float32) + ban_ref[...]
    ld_ref[...] = jnp.sum(x * wped_ref[...], axis=1, keepdims=True) + bpe_ref[...]
    ls_ref[...] = jnp.sum(x * wpes_ref[...], axis=1, keepdims=True)


def _kron_body(a_ref, b_ref, ra_ref, rb_ref, w_ref, bpk_ref, g2_ref, b2_ref, ke_ref):
    a = a_ref[...]
    b = b_ref[...]
    arep = jnp.dot(a, ra_ref[...], preferred_element_type=jnp.float32)
    brep = jnp.dot(b, rb_ref[...], preferred_element_type=jnp.float32)
    k3 = (arep * brep).astype(jnp.bfloat16)
    pre = jnp.dot(k3, w_ref[...], preferred_element_type=jnp.float32) + bpk_ref[...]
    ke_ref[...] = jnp.maximum(_ln_lanes(pre, g2_ref[...], b2_ref[...], DD), 0.0)


def _fin_body(x_ref, n0_ref, n1_ref, s0_ref, s1_ref, k0_ref, k1_ref,
              wih_ref, bih_ref, whh_ref, bhh_ref, g3_ref, b3_ref,
              wch_ref, wck_ref, bcp_ref, g4_ref, b4_ref, out_ref):
    x = x_ref[...]
    s = jnp.maximum(s0_ref[...] + s1_ref[...], 1e-12)
    ctx = jnp.maximum((n0_ref[...] + n1_ref[...]) / s, 0.0)
    nk = k0_ref[...] + k1_ref[...]
    gi = jnp.dot(ctx, wih_ref[...], preferred_element_type=jnp.float32) + bih_ref[...]
    gh = jnp.dot(x, whh_ref[...], preferred_element_type=jnp.float32) + bhh_ref[...]
    r = jax.nn.sigmoid(gi[:, :DD] + gh[:, :DD])
    z = jax.nn.sigmoid(gi[:, DD:2 * DD] + gh[:, DD:2 * DD])
    nc = jnp.tanh(gi[:, 2 * DD:] + r * gh[:, 2 * DD:])
    h = jnp.maximum((1.0 - z) * nc + z * x, 0.0)
    h = _ln_lanes(h, g3_ref[...], b3_ref[...], DD)
    o = (jnp.dot(h, wch_ref[...], preferred_element_type=jnp.float32)
         + jnp.dot(nk, wck_ref[...], preferred_element_type=jnp.float32) + bcp_ref[...])
    out_ref[...] = jnp.maximum(_ln_lanes(o, g4_ref[...], b4_ref[...], DD), 0.0)


def _ab_sc_body(proj_hbm, src_hbm, dst_hbm, a_hbm, b_hbm,
                idxs_v, idxd_v, ag_v, bg_v, sem_a, sem_b):
    c = lax.axis_index("c")
    sid = lax.axis_index("s")
    wid = sid * NC + c
    base_e = wid * EW
    gsz = GS * CH

    def group(gq, carry):
        pltpu.sync_copy(src_hbm.at[pl.ds(base_e + gq * gsz, gsz)], idxs_v)
        pltpu.sync_copy(dst_hbm.at[pl.ds(base_e + gq * gsz, gsz)], idxd_v)

        def fire(jj, carry2):
            pltpu.async_copy(proj_hbm.at[idxs_v.at[pl.ds(jj * CH, CH)]],
                             ag_v.at[pl.ds(jj * CH, CH)], sem_a)
            pltpu.async_copy(proj_hbm.at[idxd_v.at[pl.ds(jj * CH, CH)]],
                             bg_v.at[pl.ds(jj * CH, CH)], sem_b)
            return carry2

        lax.fori_loop(0, GS, fire, 0, unroll=False)
        dst_a = a_hbm.at[pl.ds(base_e + gq * gsz, gsz)]
        dst_b = b_hbm.at[pl.ds(base_e + gq * gsz, gsz)]
        pltpu.make_async_copy(dst_a, ag_v, sem_a).wait()
        pltpu.make_async_copy(dst_b, bg_v, sem_b).wait()
        pltpu.sync_copy(ag_v, dst_a)
        pltpu.sync_copy(bg_v, dst_b)
        return carry

    lax.fori_loop(0, NG, group, 0, unroll=False)


def _att_sc_body(hv_hbm, ld_hbm, ls_hbm, src_hbm, dst_hbm, zero_hbm,
                 num_hbm, s_hbm,
                 ld_v, ls_v, idxs_v, idxd_v, idxd2_v, e_v, hs_v, sem_h,
                 num_sh, s_sh):
    c = lax.axis_index("c")
    sid = lax.axis_index("s")
    wid = sid * NC + c
    base_e = wid * EW
    base_n = sid * SLICE

    pltpu.sync_copy(zero_hbm.at[pl.ds(base_n, SLICE)], num_sh.at[pl.ds(base_n, SLICE)])
    pltpu.sync_copy(zero_hbm.at[0, pl.ds(0, SLICE)], s_sh.at[pl.ds(base_n, SLICE)])

    pltpu.sync_copy(ld_hbm, ld_v)
    pltpu.sync_copy(ls_hbm, ls_v)
    plsc.subcore_barrier()

    def group(gq, carry):
        gsz = GS * CH
        pltpu.sync_copy(src_hbm.at[pl.ds(base_e + gq * gsz, gsz)], idxs_v)
        pltpu.sync_copy(dst_hbm.at[pl.ds(base_e + gq * gsz, gsz)], idxd_v)

        def mk2d(jj, carry2):
            for g in range(CH // 16):
                idxd2_v[jj, pl.ds(g * 16, 16)] = idxd_v[pl.ds(jj * CH + g * 16, 16)]
            return carry2

        lax.fori_loop(0, GS, mk2d, 0, unroll=False)
        pltpu.async_copy(hv_hbm.at[idxs_v.at[pl.ds(0, CH)]], hs_v.at[0], sem_h)

        def chunk(jj, carry2):
            p = jax.lax.rem(jj, 2)
            idx_d = idxd2_v.at[jj]
            pltpu.make_async_copy(hv_hbm.at[idxs_v.at[pl.ds(jj * CH, CH)]],
                                  hs_v.at[p], sem_h).wait()

            @pl.when(jj + 1 < GS)
            def _prefetch():
                pltpu.async_copy(hv_hbm.at[idxs_v.at[pl.ds((jj + 1) * CH, CH)]],
                                 hs_v.at[1 - p], sem_h)

            for g in range(CH // 16):
                ids = idxs_v[pl.ds(jj * CH + g * 16, 16)]
                idd = idxd_v[pl.ds(jj * CH + g * 16, 16)]
                lsg = plsc.load_gather(ls_v, [ids])
                ldg = plsc.load_gather(ld_v, [idd])
                e_v[pl.ds(g * 16, 16)] = jnp.exp(jnp.maximum(ldg + lsg, 0.0))

            def scale_row(r, carry3):
                w16 = plsc.load_gather(e_v, [jnp.full((16,), r, jnp.int32)])
                for q in range(DD // 16):
                    hs_v[p, r, pl.ds(q * 16, 16)] = hs_v[p, r, pl.ds(q * 16, 16)] * w16
                return carry3

            lax.fori_loop(0, CH, scale_row, 0, unroll=False)
            pltpu.sync_copy(hs_v.at[p], num_sh.at[idx_d], add=True)
            pltpu.sync_copy(e_v, s_sh.at[idx_d], add=True)
            return carry2

        lax.fori_loop(0, GS, chunk, 0, unroll=False)
        return carry

    lax.fori_loop(0, NG, group, 0, unroll=False)
    plsc.subcore_barrier()
    pltpu.sync_copy(num_sh.at[pl.ds(base_n, SLICE)], num_hbm.at[c, pl.ds(base_n, SLICE)])
    pltpu.sync_copy(s_sh.at[pl.ds(base_n, SLICE)], s_hbm.at[c, pl.ds(base_n, SLICE)])


def _nk_sc_body(ke_hbm, dst_hbm, zero_hbm, nk_hbm, idxd_v, idxd2_v, ke_v, sem_k, nk_sh):
    c = lax.axis_index("c")
    sid = lax.axis_index("s")
    wid = sid * NC + c
    base_e = wid * EW
    base_n = sid * SLICE

    pltpu.sync_copy(zero_hbm.at[pl.ds(base_n, SLICE)], nk_sh.at[pl.ds(base_n, SLICE)])
    pltpu.sync_copy(dst_hbm.at[pl.ds(base_e, EW)], idxd_v)

    def mk2d(jj, carry2):
        for g in range(CH // 16):
            idxd2_v[jj, pl.ds(g * 16, 16)] = idxd_v[pl.ds(jj * CH + g * 16, 16)]
        return carry2

    lax.fori_loop(0, NCH, mk2d, 0, unroll=False)
    plsc.subcore_barrier()
    pltpu.async_copy(ke_hbm.at[pl.ds(base_e, CH)], ke_v.at[0], sem_k)

    def chunk(j, carry):
        p = jax.lax.rem(j, 2)
        pltpu.make_async_copy(ke_hbm.at[pl.ds(base_e + j * CH, CH)], ke_v.at[p], sem_k).wait()

        @pl.when(j + 1 < NCH)
        def _prefetch():
            pltpu.async_copy(ke_hbm.at[pl.ds(base_e + (j + 1) * CH, CH)], ke_v.at[1 - p], sem_k)

        pltpu.sync_copy(ke_v.at[p], nk_sh.at[idxd2_v.at[j]], add=True)
        return carry

    lax.fori_loop(0, NCH, chunk, 0, unroll=False)
    plsc.subcore_barrier()
    pltpu.sync_copy(nk_sh.at[pl.ds(base_n, SLICE)], nk_hbm.at[c, pl.ds(base_n, SLICE)])


_SC_MESH = plsc.VectorSubcoreMesh(core_axis_name="c", subcore_axis_name="s",
                                  num_cores=NC, num_subcores=NS)
_SC_PARAMS = pltpu.CompilerParams(use_tc_tiling_on_sc=False,
                                  needs_layout_passes=False)


def _make_ab_sc():
    f32 = jnp.float32
    return functools.partial(
        pl.kernel, mesh=_SC_MESH, compiler_params=_SC_PARAMS,
        out_type=(jax.ShapeDtypeStruct((EE, 32), jnp.bfloat16),
                  jax.ShapeDtypeStruct((EE, 32), jnp.bfloat16)),
        scratch_types=[pltpu.VMEM((GS * CH,), jnp.int32), pltpu.VMEM((GS * CH,), jnp.int32),
                       pltpu.VMEM((GS * CH, 32), jnp.bfloat16),
                       pltpu.VMEM((GS * CH, 32), jnp.bfloat16),
                       pltpu.SemaphoreType.DMA, pltpu.SemaphoreType.DMA],
    )(_ab_sc_body)


def _make_att_sc():
    f32 = jnp.float32
    return functools.partial(
        pl.kernel, mesh=_SC_MESH, compiler_params=_SC_PARAMS,
        out_type=(jax.ShapeDtypeStruct((NC, NP, DD), f32),
                  jax.ShapeDtypeStruct((NC, NP), f32)),
        scratch_types=[pltpu.VMEM((NN,), f32), pltpu.VMEM((NN,), f32),
                       pltpu.VMEM((GS * CH,), jnp.int32), pltpu.VMEM((GS * CH,), jnp.int32),
                       pltpu.VMEM((GS, CH), jnp.int32),
                       pltpu.VMEM((CH,), f32), pltpu.VMEM((2, CH, DD), f32),
                       pltpu.SemaphoreType.DMA,
                       pltpu.VMEM_SHARED((NP, DD), f32), pltpu.VMEM_SHARED((NP,), f32)],
    )(_att_sc_body)


def _make_nk_sc():
    f32 = jnp.float32
    return functools.partial(
        pl.kernel, mesh=_SC_MESH, compiler_params=_SC_PARAMS,
        out_type=jax.ShapeDtypeStruct((NC, NP, DD), f32),
        scratch_types=[pltpu.VMEM((EW,), jnp.int32), pltpu.VMEM((NCH, CH), jnp.int32),
                       pltpu.VMEM((2, CH, DD), f32),
                       pltpu.SemaphoreType.DMA,
                       pltpu.VMEM_SHARED((NP, DD), f32)],
    )(_nk_sc_body)


def kernel(x, edge_index, W_pn, b_pn, ln1_g, ln1_b, W_pk, b_pk, ln2_g, ln2_b,
           W_pe, b_pe, W_an, b_an, W_ih, b_ih, W_hh, b_hh, ln3_g, ln3_b,
           W_cp, b_cp, ln4_g, ln4_b):
    f32 = jnp.float32
    wpn32 = jnp.pad(W_pn, ((0, 0), (0, 12)))
    bpn32 = jnp.pad(b_pn, (0, 12)).reshape(1, 32)
    g1 = jnp.pad(ln1_g, (0, 12)).reshape(1, 32)
    b1 = jnp.pad(ln1_b, (0, 12)).reshape(1, 32)
    w640 = jnp.pad(W_pk.reshape(KP, KP, DD), ((0, 0), (0, 12), (0, 0))).reshape(640, DD).astype(jnp.bfloat16)
    wped = W_pe[:DD].reshape(1, DD)
    wpes = W_pe[DD:].reshape(1, DD)
    bpe = b_pe.reshape(1, 1)
    src1 = edge_index[0]
    dst1 = edge_index[1]
    zero_np = jnp.zeros((NP, DD), f32)
    ii = _np.arange(32)[:, None]
    jj = _np.arange(640)[None, :]
    ra = jnp.asarray((jj // 32 == ii), dtype=jnp.bfloat16)
    rb = jnp.asarray((jj % 32 == ii), dtype=jnp.bfloat16)

    full2 = lambda shp: pl.BlockSpec(shp, lambda i: (0, 0))

    proj, hv, ld, ls = pl.pallas_call(
        _pre_body,
        grid=(NN // BN,),
        in_specs=[pl.BlockSpec((BN, DD), lambda i: (i, 0)),
                  full2((DD, 32)), full2((1, 32)), full2((1, 32)), full2((1, 32)),
                  full2((DD, DD)), full2((1, DD)), full2((1, DD)), full2((1, DD)),
                  full2((1, 1))],
        out_specs=[pl.BlockSpec((BN, 32), lambda i: (i, 0)),
                   pl.BlockSpec((BN, DD), lambda i: (i, 0)),
                   pl.BlockSpec((BN, 1), lambda i: (i, 0)),
                   pl.BlockSpec((BN, 1), lambda i: (i, 0))],
        out_shape=[jax.ShapeDtypeStruct((NN, 32), jnp.bfloat16),
                   jax.ShapeDtypeStruct((NN, DD), f32),
                   jax.ShapeDtypeStruct((NN, 1), f32),
                   jax.ShapeDtypeStruct((NN, 1), f32)],
    )(x, wpn32, bpn32, g1, b1, W_an, b_an.reshape(1, DD), wped, wpes, bpe)

    ld1 = ld.reshape(NN)
    ls1 = ls.reshape(NN)

    a_arr, b_arr = _make_ab_sc()(proj, src1, dst1)
    num_p, s_p = _make_att_sc()(hv, ld1, ls1, src1, dst1, zero_np)

    ke = pl.pallas_call(
        _kron_body,
        grid=(EE // BE,),
        in_specs=[pl.BlockSpec((BE, 32), lambda i: (i, 0)),
                  pl.BlockSpec((BE, 32), lambda i: (i, 0)),
                  full2((32, 640)), full2((32, 640)),
                  full2((640, DD)), full2((1, DD)), full2((1, DD)), full2((1, DD))],
        out_specs=pl.BlockSpec((BE, DD), lambda i: (i, 0)),
        out_shape=jax.ShapeDtypeStruct((EE, DD), f32),
    )(a_arr, b_arr, ra, rb, w640, b_pk.reshape(1, DD), ln2_g.reshape(1, DD), ln2_b.reshape(1, DD))

    nk_p = _make_nk_sc()(ke, dst1, zero_np)

    out = pl.pallas_call(
        _fin_body,
        grid=(NN // BN,),
        in_specs=[pl.BlockSpec((BN, DD), lambda i: (i, 0)),
                  pl.BlockSpec((BN, DD), lambda i: (i, 0)),
                  pl.BlockSpec((BN, DD), lambda i: (i, 0)),
                  pl.BlockSpec((BN, 1), lambda i: (i, 0)),
                  pl.BlockSpec((BN, 1), lambda i: (i, 0)),
                  pl.BlockSpec((BN, DD), lambda i: (i, 0)),
                  pl.BlockSpec((BN, DD), lambda i: (i, 0)),
                  full2((DD, 3 * DD)), full2((1, 3 * DD)),
                  full2((DD, 3 * DD)), full2((1, 3 * DD)),
                  full2((1, DD)), full2((1, DD)),
                  full2((DD, DD)), full2((DD, DD)), full2((1, DD)),
                  full2((1, DD)), full2((1, DD))],
        out_specs=pl.BlockSpec((BN, DD), lambda i: (i, 0)),
        out_shape=jax.ShapeDtypeStruct((NN, DD), f32),
    )(x, num_p[0, :NN], num_p[1, :NN], s_p[0, :NN].reshape(NN, 1), s_p[1, :NN].reshape(NN, 1),
      nk_p[0, :NN], nk_p[1, :NN],
      W_ih.T, b_ih.reshape(1, 3 * DD), W_hh.T, b_hh.reshape(1, 3 * DD),
      ln3_g.reshape(1, DD), ln3_b.reshape(1, DD),
      W_cp[:DD], W_cp[DD:], b_cp.reshape(1, DD),
      ln4_g.reshape(1, DD), ln4_b.reshape(1, DD))
    return out


# speedup vs baseline: 9.1539x; 1.0387x over previous
import functools

import numpy as _np

import jax
import jax.numpy as jnp
from jax import lax
from jax.experimental import pallas as pl
from jax.experimental.pallas import tpu as pltpu
from jax.experimental.pallas import tpu_sc as plsc

NN = 10000
NP = 10240
EE = 320000
DD = 128
KP = 20
NC = 2
NS = 16
NW = NC * NS
EW = EE // NW
CH = 80
NCH = EW // CH
GS = 25
NG = NCH // GS
SLICE = NP // NS
BN = 1000
BE = 2560

_EPS = 1e-5


def _ln_lanes(y, g, b, n_lanes):
    mu = jnp.sum(y, axis=1, keepdims=True) / n_lanes
    diff = y - mu
    var = jnp.sum(diff * diff, axis=1, keepdims=True) / n_lanes
    return diff * lax.rsqrt(var + _EPS) * g + b


def _pre_body(x_ref, wpn_ref, bpn_ref, g1_ref, b1_ref, wan_ref, ban_ref,
              wped_ref, wpes_ref, bpe_ref, proj_ref, hv_ref, ld_ref, ls_ref):
    x = x_ref[...]
    p = jnp.dot(x, wpn_ref[...], preferred_element_type=jnp.float32) + bpn_ref[...]
    mask = lax.broadcasted_iota(jnp.int32, p.shape, 1) < KP
    mu = jnp.sum(p, axis=1, keepdims=True) / KP
    diff = jnp.where(mask, p - mu, 0.0)
    var = jnp.sum(diff * diff, axis=1, keepdims=True) / KP
    proj_ref[...] = jnp.maximum(diff * lax.rsqrt(var + _EPS) * g1_ref[...] + b1_ref[...],
                                0.0).astype(jnp.bfloat16)
    hv_ref[...] = jnp.dot(x, wan_ref[...], preferred_element_type=jnp.float32) + ban_ref[...]
    ld_ref[...] = jnp.sum(x * wped_ref[...], axis=1, keepdims=True) + bpe_ref[...]
    ls_ref[...] = jnp.sum(x * wpes_ref[...], axis=1, keepdims=True)


def _kron_body(a_ref, b_ref, ra_ref, rb_ref, w_ref, bpk_ref, g2_ref, b2_ref, ke_ref):
    a = a_ref[...]
    b = b_ref[...]
    arep = jnp.dot(a, ra_ref[...], preferred_element_type=jnp.float32)
    brep = jnp.dot(b, rb_ref[...], preferred_element_type=jnp.float32)
    k3 = (arep * brep).astype(jnp.bfloat16)
    pre = jnp.dot(k3, w_ref[...], preferred_element_type=jnp.float32) + bpk_ref[...]
    ke_ref[...] = jnp.maximum(_ln_lanes(pre, g2_ref[...], b2_ref[...], DD), 0.0)


def _fin_body(x_ref, n0_ref, n1_ref, s0_ref, s1_ref, k0_ref, k1_ref, k2_ref, k3_ref,
              wih_ref, bih_ref, whh_ref, bhh_ref, g3_ref, b3_ref,
              wch_ref, wck_ref, bcp_ref, g4_ref, b4_ref, out_ref):
    x = x_ref[...]
    s = jnp.maximum(s0_ref[...] + s1_ref[...], 1e-12)
    ctx = jnp.maximum((n0_ref[...] + n1_ref[...]) / s, 0.0)
    nk = (k0_ref[...] + k1_ref[...]) + (k2_ref[...] + k3_ref[...])
    gi = jnp.dot(ctx, wih_ref[...], preferred_element_type=jnp.float32) + bih_ref[...]
    gh = jnp.dot(x, whh_ref[...], preferred_element_type=jnp.float32) + bhh_ref[...]
    r = jax.nn.sigmoid(gi[:, :DD] + gh[:, :DD])
    z = jax.nn.sigmoid(gi[:, DD:2 * DD] + gh[:, DD:2 * DD])
    nc = jnp.tanh(gi[:, 2 * DD:] + r * gh[:, 2 * DD:])
    h = jnp.maximum((1.0 - z) * nc + z * x, 0.0)
    h = _ln_lanes(h, g3_ref[...], b3_ref[...], DD)
    o = (jnp.dot(h, wch_ref[...], preferred_element_type=jnp.float32)
         + jnp.dot(nk, wck_ref[...], preferred_element_type=jnp.float32) + bcp_ref[...])
    out_ref[...] = jnp.maximum(_ln_lanes(o, g4_ref[...], b4_ref[...], DD), 0.0)


def _ab_sc_body(proj_hbm, src_hbm, dst_hbm, a_hbm, b_hbm,
                idxs_v, idxd_v, ag_v, bg_v, sem_a, sem_b):
    c = lax.axis_index("c")
    sid = lax.axis_index("s")
    wid = sid * NC + c
    base_e = wid * EW
    gsz = GS * CH

    def group(gq, carry):
        pltpu.sync_copy(src_hbm.at[pl.ds(base_e + gq * gsz, gsz)], idxs_v)
        pltpu.sync_copy(dst_hbm.at[pl.ds(base_e + gq * gsz, gsz)], idxd_v)

        def fire(jj, carry2):
            pltpu.async_copy(proj_hbm.at[idxs_v.at[pl.ds(jj * CH, CH)]],
                             ag_v.at[pl.ds(jj * CH, CH)], sem_a)
            pltpu.async_copy(proj_hbm.at[idxd_v.at[pl.ds(jj * CH, CH)]],
                             bg_v.at[pl.ds(jj * CH, CH)], sem_b)
            return carry2

        lax.fori_loop(0, GS, fire, 0, unroll=False)
        dst_a = a_hbm.at[pl.ds(base_e + gq * gsz, gsz)]
        dst_b = b_hbm.at[pl.ds(base_e + gq * gsz, gsz)]
        pltpu.make_async_copy(dst_a, ag_v, sem_a).wait()
        pltpu.make_async_copy(dst_b, bg_v, sem_b).wait()
        pltpu.sync_copy(ag_v, dst_a)
        pltpu.sync_copy(bg_v, dst_b)
        return carry

    lax.fori_loop(0, NG, group, 0, unroll=False)


def _att_sc_body(hv_hbm, ld_hbm, ls_hbm, src_hbm, dst_hbm, zero_hbm,
                 num_hbm, s_hbm,
                 ld_v, ls_v, idxs_v, idxd_v, idxd2_v, e_v, hs_v, sem_h,
                 num_sh, s_sh):
    c = lax.axis_index("c")
    sid = lax.axis_index("s")
    wid = sid * NC + c
    base_e = wid * EW
    base_n = sid * SLICE

    pltpu.sync_copy(zero_hbm.at[pl.ds(base_n, SLICE)], num_sh.at[pl.ds(base_n, SLICE)])
    pltpu.sync_copy(zero_hbm.at[0, pl.ds(0, SLICE)], s_sh.at[pl.ds(base_n, SLICE)])

    pltpu.sync_copy(ld_hbm, ld_v)
    pltpu.sync_copy(ls_hbm, ls_v)
    plsc.subcore_barrier()

    def group(gq, carry):
        gsz = GS * CH
        pltpu.sync_copy(src_hbm.at[pl.ds(base_e + gq * gsz, gsz)], idxs_v)
        pltpu.sync_copy(dst_hbm.at[pl.ds(base_e + gq * gsz, gsz)], idxd_v)

        def mk2d(jj, carry2):
            for g in range(CH // 16):
                idxd2_v[jj, pl.ds(g * 16, 16)] = idxd_v[pl.ds(jj * CH + g * 16, 16)]
            return carry2

        lax.fori_loop(0, GS, mk2d, 0, unroll=False)
        pltpu.async_copy(hv_hbm.at[idxs_v.at[pl.ds(0, CH)]], hs_v.at[0], sem_h)

        def chunk(jj, carry2):
            p = jax.lax.rem(jj, 2)
            idx_d = idxd2_v.at[jj]
            pltpu.make_async_copy(hv_hbm.at[idxs_v.at[pl.ds(jj * CH, CH)]],
                                  hs_v.at[p], sem_h).wait()

            @pl.when(jj + 1 < GS)
            def _prefetch():
                pltpu.async_copy(hv_hbm.at[idxs_v.at[pl.ds((jj + 1) * CH, CH)]],
                                 hs_v.at[1 - p], sem_h)

            for g in range(CH // 16):
                ids = idxs_v[pl.ds(jj * CH + g * 16, 16)]
                idd = idxd_v[pl.ds(jj * CH + g * 16, 16)]
                lsg = plsc.load_gather(ls_v, [ids])
                ldg = plsc.load_gather(ld_v, [idd])
                e_v[pl.ds(g * 16, 16)] = jnp.exp(jnp.maximum(ldg + lsg, 0.0))

            def scale_row(r, carry3):
                w16 = plsc.load_gather(e_v, [jnp.full((16,), r, jnp.int32)])
                for q in range(DD // 16):
                    hs_v[p, r, pl.ds(q * 16, 16)] = hs_v[p, r, pl.ds(q * 16, 16)] * w16
                return carry3

            lax.fori_loop(0, CH, scale_row, 0, unroll=False)
            pltpu.sync_copy(hs_v.at[p], num_sh.at[idx_d], add=True)
            pltpu.sync_copy(e_v, s_sh.at[idx_d], add=True)
            return carry2

        lax.fori_loop(0, GS, chunk, 0, unroll=False)
        return carry

    lax.fori_loop(0, NG, group, 0, unroll=False)
    plsc.subcore_barrier()
    pltpu.sync_copy(num_sh.at[pl.ds(base_n, SLICE)], num_hbm.at[c, pl.ds(base_n, SLICE)])
    pltpu.sync_copy(s_sh.at[pl.ds(base_n, SLICE)], s_hbm.at[c, pl.ds(base_n, SLICE)])


def _mk_nk_body(ew, nch):
    def _nk_sc_body(ke_hbm, dst_hbm, zero_hbm, nk_hbm, idxd_v, idxd2_v, ke_v, sem_k, nk_sh):
        c = lax.axis_index("c")
        sid = lax.axis_index("s")
        wid = sid * NC + c
        base_e = wid * ew
        base_n = sid * SLICE

        pltpu.sync_copy(zero_hbm.at[pl.ds(base_n, SLICE)], nk_sh.at[pl.ds(base_n, SLICE)])
        pltpu.sync_copy(dst_hbm.at[pl.ds(base_e, ew)], idxd_v)

        def mk2d(jj, carry2):
            for g in range(CH // 16):
                idxd2_v[jj, pl.ds(g * 16, 16)] = idxd_v[pl.ds(jj * CH + g * 16, 16)]
            return carry2

        lax.fori_loop(0, nch, mk2d, 0, unroll=False)
        plsc.subcore_barrier()
        pltpu.async_copy(ke_hbm.at[pl.ds(base_e, CH)], ke_v.at[0], sem_k)

        def chunk(j, carry):
            p = jax.lax.rem(j, 2)
            pltpu.make_async_copy(ke_hbm.at[pl.ds(base_e + j * CH, CH)], ke_v.at[p], sem_k).wait()

            @pl.when(j + 1 < nch)
            def _prefetch():
                pltpu.async_copy(ke_hbm.at[pl.ds(base_e + (j + 1) * CH, CH)], ke_v.at[1 - p],
                                 sem_k)

            pltpu.sync_copy(ke_v.at[p], nk_sh.at[idxd2_v.at[j]], add=True)
            return carry

        lax.fori_loop(0, nch, chunk, 0, unroll=False)
        plsc.subcore_barrier()
        pltpu.sync_copy(nk_sh.at[pl.ds(base_n, SLICE)], nk_hbm.at[c, pl.ds(base_n, SLICE)])
    return _nk_sc_body


_SC_MESH = plsc.VectorSubcoreMesh(core_axis_name="c", subcore_axis_name="s",
                                  num_cores=NC, num_subcores=NS)
_SC_PARAMS = pltpu.CompilerParams(use_tc_tiling_on_sc=False,
                                  needs_layout_passes=False)


def _make_ab_sc():
    f32 = jnp.float32
    return functools.partial(
        pl.kernel, mesh=_SC_MESH, compiler_params=_SC_PARAMS,
        out_type=(jax.ShapeDtypeStruct((EE, 32), jnp.bfloat16),
                  jax.ShapeDtypeStruct((EE, 32), jnp.bfloat16)),
        scratch_types=[pltpu.VMEM((GS * CH,), jnp.int32), pltpu.VMEM((GS * CH,), jnp.int32),
                       pltpu.VMEM((GS * CH, 32), jnp.bfloat16),
                       pltpu.VMEM((GS * CH, 32), jnp.bfloat16),
                       pltpu.SemaphoreType.DMA, pltpu.SemaphoreType.DMA],
    )(_ab_sc_body)


def _make_att_sc():
    f32 = jnp.float32
    return functools.partial(
        pl.kernel, mesh=_SC_MESH, compiler_params=_SC_PARAMS,
        out_type=(jax.ShapeDtypeStruct((NC, NP, DD), f32),
                  jax.ShapeDtypeStruct((NC, NP), f32)),
        scratch_types=[pltpu.VMEM((NN,), f32), pltpu.VMEM((NN,), f32),
                       pltpu.VMEM((GS * CH,), jnp.int32), pltpu.VMEM((GS * CH,), jnp.int32),
                       pltpu.VMEM((GS, CH), jnp.int32),
                       pltpu.VMEM((CH,), f32), pltpu.VMEM((2, CH, DD), f32),
                       pltpu.SemaphoreType.DMA,
                       pltpu.VMEM_SHARED((NP, DD), f32), pltpu.VMEM_SHARED((NP,), f32)],
    )(_att_sc_body)


def _make_nk_sc(ew, nch):
    f32 = jnp.float32
    return functools.partial(
        pl.kernel, mesh=_SC_MESH, compiler_params=_SC_PARAMS,
        out_type=jax.ShapeDtypeStruct((NC, NP, DD), f32),
        scratch_types=[pltpu.VMEM((ew,), jnp.int32), pltpu.VMEM((nch, CH), jnp.int32),
                       pltpu.VMEM((2, CH, DD), f32),
                       pltpu.SemaphoreType.DMA,
                       pltpu.VMEM_SHARED((NP, DD), f32)],
    )(_mk_nk_body(ew, nch))


def kernel(x, edge_index, W_pn, b_pn, ln1_g, ln1_b, W_pk, b_pk, ln2_g, ln2_b,
           W_pe, b_pe, W_an, b_an, W_ih, b_ih, W_hh, b_hh, ln3_g, ln3_b,
           W_cp, b_cp, ln4_g, ln4_b):
    f32 = jnp.float32
    wpn32 = jnp.pad(W_pn, ((0, 0), (0, 12)))
    bpn32 = jnp.pad(b_pn, (0, 12)).reshape(1, 32)
    g1 = jnp.pad(ln1_g, (0, 12)).reshape(1, 32)
    b1 = jnp.pad(ln1_b, (0, 12)).reshape(1, 32)
    w640 = jnp.pad(W_pk.reshape(KP, KP, DD), ((0, 0), (0, 12), (0, 0))).reshape(640, DD).astype(jnp.bfloat16)
    wped = W_pe[:DD].reshape(1, DD)
    wpes = W_pe[DD:].reshape(1, DD)
    bpe = b_pe.reshape(1, 1)
    src1 = edge_index[0]
    dst1 = edge_index[1]
    zero_np = jnp.zeros((NP, DD), f32)
    ii = _np.arange(32)[:, None]
    jj = _np.arange(640)[None, :]
    ra = jnp.asarray((jj // 32 == ii), dtype=jnp.bfloat16)
    rb = jnp.asarray((jj % 32 == ii), dtype=jnp.bfloat16)

    full2 = lambda shp: pl.BlockSpec(shp, lambda i: (0, 0))

    proj, hv, ld, ls = pl.pallas_call(
        _pre_body,
        grid=(NN // BN,),
        in_specs=[pl.BlockSpec((BN, DD), lambda i: (i, 0)),
                  full2((DD, 32)), full2((1, 32)), full2((1, 32)), full2((1, 32)),
                  full2((DD, DD)), full2((1, DD)), full2((1, DD)), full2((1, DD)),
                  full2((1, 1))],
        out_specs=[pl.BlockSpec((BN, 32), lambda i: (i, 0)),
                   pl.BlockSpec((BN, DD), lambda i: (i, 0)),
                   pl.BlockSpec((BN, 1), lambda i: (i, 0)),
                   pl.BlockSpec((BN, 1), lambda i: (i, 0))],
        out_shape=[jax.ShapeDtypeStruct((NN, 32), jnp.bfloat16),
                   jax.ShapeDtypeStruct((NN, DD), f32),
                   jax.ShapeDtypeStruct((NN, 1), f32),
                   jax.ShapeDtypeStruct((NN, 1), f32)],
    )(x, wpn32, bpn32, g1, b1, W_an, b_an.reshape(1, DD), wped, wpes, bpe)

    ld1 = ld.reshape(NN)
    ls1 = ls.reshape(NN)

    a_arr, b_arr = _make_ab_sc()(proj, src1, dst1)
    num_p, s_p = _make_att_sc()(hv, ld1, ls1, src1, dst1, zero_np)

    EA = 192000
    nblk_a = EA // BE

    def kron_call(nblk, blk_off, row_off):
        return pl.pallas_call(
            _kron_body,
            grid=(nblk,),
            in_specs=[pl.BlockSpec((BE, 32), lambda i: (i + blk_off, 0)),
                      pl.BlockSpec((BE, 32), lambda i: (i + blk_off, 0)),
                      full2((32, 640)), full2((32, 640)),
                      full2((640, DD)), full2((1, DD)), full2((1, DD)), full2((1, DD))],
            out_specs=pl.BlockSpec((BE, DD), lambda i: (i, 0)),
            out_shape=jax.ShapeDtypeStruct((nblk * BE, DD), f32),
        )(a_arr, b_arr, ra, rb, w640, b_pk.reshape(1, DD), ln2_g.reshape(1, DD),
          ln2_b.reshape(1, DD))

    ke_a = kron_call(nblk_a, 0, 0)
    nk_pa = _make_nk_sc(EA // NW, EA // NW // CH)(ke_a, dst1[:EA], zero_np)
    ke_b = kron_call(EE // BE - nblk_a, nblk_a, EA)
    nk_pb = _make_nk_sc((EE - EA) // NW, (EE - EA) // NW // CH)(ke_b, dst1[EA:], zero_np)

    out = pl.pallas_call(
        _fin_body,
        grid=(NN // BN,),
        in_specs=[pl.BlockSpec((BN, DD), lambda i: (i, 0)),
                  pl.BlockSpec((BN, DD), lambda i: (i, 0)),
                  pl.BlockSpec((BN, DD), lambda i: (i, 0)),
                  pl.BlockSpec((BN, 1), lambda i: (i, 0)),
                  pl.BlockSpec((BN, 1), lambda i: (i, 0)),
                  pl.BlockSpec((BN, DD), lambda i: (i, 0)),
                  pl.BlockSpec((BN, DD), lambda i: (i, 0)),
                  pl.BlockSpec((BN, DD), lambda i: (i, 0)),
                  pl.BlockSpec((BN, DD), lambda i: (i, 0)),
                  full2((DD, 3 * DD)), full2((1, 3 * DD)),
                  full2((DD, 3 * DD)), full2((1, 3 * DD)),
                  full2((1, DD)), full2((1, DD)),
                  full2((DD, DD)), full2((DD, DD)), full2((1, DD)),
                  full2((1, DD)), full2((1, DD))],
        out_specs=pl.BlockSpec((BN, DD), lambda i: (i, 0)),
        out_shape=jax.ShapeDtypeStruct((NN, DD), f32),
    )(x, num_p[0, :NN], num_p[1, :NN], s_p[0, :NN].reshape(NN, 1), s_p[1, :NN].reshape(NN, 1),
      nk_pa[0, :NN], nk_pa[1, :NN], nk_pb[0, :NN], nk_pb[1, :NN],
      W_ih.T, b_ih.reshape(1, 3 * DD), W_hh.T, b_hh.reshape(1, 3 * DD),
      ln3_g.reshape(1, DD), ln3_b.reshape(1, DD),
      W_cp[:DD], W_cp[DD:], b_cp.reshape(1, DD),
      ln4_g.reshape(1, DD), ln4_b.reshape(1, DD))
    return out


# speedup vs baseline: 9.9060x; 1.0822x over previous
import functools

import numpy as _np

import jax
import jax.numpy as jnp
from jax import lax
from jax.experimental import pallas as pl
from jax.experimental.pallas import tpu as pltpu
from jax.experimental.pallas import tpu_sc as plsc

NN = 10000
NP = 10240
EE = 320000
DD = 128
KP = 20
NC = 2
NS = 16
NW = NC * NS
EW = EE // NW
CH = 80
NCH = EW // CH
GS = 25
NG = NCH // GS
SLICE = NP // NS
BN = 1000
BE = 2560

_EPS = 1e-5


def _ln_lanes(y, g, b, n_lanes):
    mu = jnp.sum(y, axis=1, keepdims=True) / n_lanes
    diff = y - mu
    var = jnp.sum(diff * diff, axis=1, keepdims=True) / n_lanes
    return diff * lax.rsqrt(var + _EPS) * g + b


def _pre_body(x_ref, wpn_ref, bpn_ref, g1_ref, b1_ref, wan_ref, ban_ref,
              wped_ref, wpes_ref, bpe_ref, proj_ref, hv_ref, ld_ref, ls_ref):
    x = x_ref[...]
    p = jnp.dot(x, wpn_ref[...], preferred_element_type=jnp.float32) + bpn_ref[...]
    mask = lax.broadcasted_iota(jnp.int32, p.shape, 1) < KP
    mu = jnp.sum(p, axis=1, keepdims=True) / KP
    diff = jnp.where(mask, p - mu, 0.0)
    var = jnp.sum(diff * diff, axis=1, keepdims=True) / KP
    proj_ref[...] = jnp.maximum(diff * lax.rsqrt(var + _EPS) * g1_ref[...] + b1_ref[...],
                                0.0).astype(jnp.bfloat16)
    hv_ref[...] = jnp.dot(x, wan_ref[...], preferred_element_type=jnp.float32) + ban_ref[...]
    ld_ref[...] = jnp.sum(x * wped_ref[...], axis=1, keepdims=True) + bpe_ref[...]
    ls_ref[...] = jnp.sum(x * wpes_ref[...], axis=1, keepdims=True)


def _kron_body(ab_ref, ra_ref, rb_ref, w_ref, bpk_ref, g2_ref, b2_ref, ke_ref):
    ab = ab_ref[...]
    arep = jnp.dot(ab, ra_ref[...], preferred_element_type=jnp.float32)
    brep = jnp.dot(ab, rb_ref[...], preferred_element_type=jnp.float32)
    k3 = (arep * brep).astype(jnp.bfloat16)
    pre = jnp.dot(k3, w_ref[...], preferred_element_type=jnp.float32) + bpk_ref[...]
    ke_ref[...] = jnp.maximum(_ln_lanes(pre, g2_ref[...], b2_ref[...], DD), 0.0)


def _fin_body(x_ref, n0_ref, n1_ref, s0_ref, s1_ref, k0_ref, k1_ref, k2_ref, k3_ref,
              wih_ref, bih_ref, whh_ref, bhh_ref, g3_ref, b3_ref,
              wch_ref, wck_ref, bcp_ref, g4_ref, b4_ref, out_ref):
    x = x_ref[...]
    s = jnp.maximum(s0_ref[...] + s1_ref[...], 1e-12)
    ctx = jnp.maximum((n0_ref[...] + n1_ref[...]) / s, 0.0)
    nk = (k0_ref[...] + k1_ref[...]) + (k2_ref[...] + k3_ref[...])
    gi = jnp.dot(ctx, wih_ref[...], preferred_element_type=jnp.float32) + bih_ref[...]
    gh = jnp.dot(x, whh_ref[...], preferred_element_type=jnp.float32) + bhh_ref[...]
    r = jax.nn.sigmoid(gi[:, :DD] + gh[:, :DD])
    z = jax.nn.sigmoid(gi[:, DD:2 * DD] + gh[:, DD:2 * DD])
    nc = jnp.tanh(gi[:, 2 * DD:] + r * gh[:, 2 * DD:])
    h = jnp.maximum((1.0 - z) * nc + z * x, 0.0)
    h = _ln_lanes(h, g3_ref[...], b3_ref[...], DD)
    o = (jnp.dot(h, wch_ref[...], preferred_element_type=jnp.float32)
         + jnp.dot(nk, wck_ref[...], preferred_element_type=jnp.float32) + bcp_ref[...])
    out_ref[...] = jnp.maximum(_ln_lanes(o, g4_ref[...], b4_ref[...], DD), 0.0)


def _ab_sc_body(proj_hbm, src_hbm, dst_hbm, ab_hbm,
                idxs_v, idxd_v, ag_v, bg_v, sem_a, sem_b):
    c = lax.axis_index("c")
    sid = lax.axis_index("s")
    wid = sid * NC + c
    base_e = wid * EW
    gsz = GS * CH

    def group(gq, carry):
        pltpu.sync_copy(src_hbm.at[pl.ds(base_e + gq * gsz, gsz)], idxs_v)
        pltpu.sync_copy(dst_hbm.at[pl.ds(base_e + gq * gsz, gsz)], idxd_v)

        def fire(jj, carry2):
            pltpu.async_copy(proj_hbm.at[idxs_v.at[pl.ds(jj * CH, CH)]],
                             ag_v.at[pl.ds(jj * CH, CH)], sem_a)
            pltpu.async_copy(proj_hbm.at[idxd_v.at[pl.ds(jj * CH, CH)]],
                             bg_v.at[pl.ds(jj * CH, CH)], sem_b)
            return carry2

        lax.fori_loop(0, GS, fire, 0, unroll=False)
        dst_a = ab_hbm.at[pl.ds(base_e + gq * gsz, gsz), pl.ds(0, 32)]
        dst_b = ab_hbm.at[pl.ds(base_e + gq * gsz, gsz), pl.ds(32, 32)]
        pltpu.make_async_copy(dst_a, ag_v, sem_a).wait()
        pltpu.make_async_copy(dst_b, bg_v, sem_b).wait()
        pltpu.sync_copy(ag_v, dst_a)
        pltpu.sync_copy(bg_v, dst_b)
        return carry

    lax.fori_loop(0, NG, group, 0, unroll=False)


def _att_sc_body(hv_hbm, ld_hbm, ls_hbm, src_hbm, dst_hbm, zero_hbm,
                 num_hbm, s_hbm,
                 ld_v, ls_v, idxs_v, idxd_v, idxd2_v, e_v, hs_v, sem_h,
                 num_sh, s_sh):
    c = lax.axis_index("c")
    sid = lax.axis_index("s")
    wid = sid * NC + c
    base_e = wid * EW
    base_n = sid * SLICE

    pltpu.sync_copy(zero_hbm.at[pl.ds(base_n, SLICE)], num_sh.at[pl.ds(base_n, SLICE)])
    pltpu.sync_copy(zero_hbm.at[0, pl.ds(0, SLICE)], s_sh.at[pl.ds(base_n, SLICE)])

    pltpu.sync_copy(ld_hbm, ld_v)
    pltpu.sync_copy(ls_hbm, ls_v)
    plsc.subcore_barrier()

    def group(gq, carry):
        gsz = GS * CH
        pltpu.sync_copy(src_hbm.at[pl.ds(base_e + gq * gsz, gsz)], idxs_v)
        pltpu.sync_copy(dst_hbm.at[pl.ds(base_e + gq * gsz, gsz)], idxd_v)

        def mk2d(jj, carry2):
            for g in range(CH // 16):
                idxd2_v[jj, pl.ds(g * 16, 16)] = idxd_v[pl.ds(jj * CH + g * 16, 16)]
            return carry2

        lax.fori_loop(0, GS, mk2d, 0, unroll=False)
        pltpu.async_copy(hv_hbm.at[idxs_v.at[pl.ds(0, CH)]], hs_v.at[0], sem_h)

        def chunk(jj, carry2):
            p = jax.lax.rem(jj, 2)
            idx_d = idxd2_v.at[jj]
            pltpu.make_async_copy(hv_hbm.at[idxs_v.at[pl.ds(jj * CH, CH)]],
                                  hs_v.at[p], sem_h).wait()

            @pl.when(jj + 1 < GS)
            def _prefetch():
                pltpu.async_copy(hv_hbm.at[idxs_v.at[pl.ds((jj + 1) * CH, CH)]],
                                 hs_v.at[1 - p], sem_h)

            for g in range(CH // 16):
                ids = idxs_v[pl.ds(jj * CH + g * 16, 16)]
                idd = idxd_v[pl.ds(jj * CH + g * 16, 16)]
                lsg = plsc.load_gather(ls_v, [ids])
                ldg = plsc.load_gather(ld_v, [idd])
                e_v[pl.ds(g * 16, 16)] = jnp.exp(jnp.maximum(ldg + lsg, 0.0))

            def scale_row(r, carry3):
                w16 = plsc.load_gather(e_v, [jnp.full((16,), r, jnp.int32)])
                for q in range(DD // 16):
                    hs_v[p, r, pl.ds(q * 16, 16)] = hs_v[p, r, pl.ds(q * 16, 16)] * w16
                return carry3

            lax.fori_loop(0, CH, scale_row, 0, unroll=False)
            pltpu.sync_copy(hs_v.at[p], num_sh.at[idx_d], add=True)
            pltpu.sync_copy(e_v, s_sh.at[idx_d], add=True)
            return carry2

        lax.fori_loop(0, GS, chunk, 0, unroll=False)
        return carry

    lax.fori_loop(0, NG, group, 0, unroll=False)
    plsc.subcore_barrier()
    pltpu.sync_copy(num_sh.at[pl.ds(base_n, SLICE)], num_hbm.at[c, pl.ds(base_n, SLICE)])
    pltpu.sync_copy(s_sh.at[pl.ds(base_n, SLICE)], s_hbm.at[c, pl.ds(base_n, SLICE)])


def _mk_nk_body(ew, nch):
    def _nk_sc_body(ke_hbm, dst_hbm, zero_hbm, nk_hbm, idxd_v, idxd2_v, ke_v, sem_k, nk_sh):
        c = lax.axis_index("c")
        sid = lax.axis_index("s")
        wid = sid * NC + c
        base_e = wid * ew
        base_n = sid * SLICE

        pltpu.sync_copy(zero_hbm.at[pl.ds(base_n, SLICE)], nk_sh.at[pl.ds(base_n, SLICE)])
        pltpu.sync_copy(dst_hbm.at[pl.ds(base_e, ew)], idxd_v)

        def mk2d(jj, carry2):
            for g in range(CH // 16):
                idxd2_v[jj, pl.ds(g * 16, 16)] = idxd_v[pl.ds(jj * CH + g * 16, 16)]
            return carry2

        lax.fori_loop(0, nch, mk2d, 0, unroll=False)
        plsc.subcore_barrier()
        pltpu.async_copy(ke_hbm.at[pl.ds(base_e, CH)], ke_v.at[0], sem_k)

        def chunk(j, carry):
            p = jax.lax.rem(j, 2)
            pltpu.make_async_copy(ke_hbm.at[pl.ds(base_e + j * CH, CH)], ke_v.at[p], sem_k).wait()

            @pl.when(j + 1 < nch)
            def _prefetch():
                pltpu.async_copy(ke_hbm.at[pl.ds(base_e + (j + 1) * CH, CH)], ke_v.at[1 - p],
                                 sem_k)

            pltpu.sync_copy(ke_v.at[p], nk_sh.at[idxd2_v.at[j]], add=True)
            return carry

        lax.fori_loop(0, nch, chunk, 0, unroll=False)
        plsc.subcore_barrier()
        pltpu.sync_copy(nk_sh.at[pl.ds(base_n, SLICE)], nk_hbm.at[c, pl.ds(base_n, SLICE)])
    return _nk_sc_body


_SC_MESH = plsc.VectorSubcoreMesh(core_axis_name="c", subcore_axis_name="s",
                                  num_cores=NC, num_subcores=NS)
_SC_PARAMS = pltpu.CompilerParams(use_tc_tiling_on_sc=False,
                                  needs_layout_passes=False)


def _make_ab_sc():
    f32 = jnp.float32
    return functools.partial(
        pl.kernel, mesh=_SC_MESH, compiler_params=_SC_PARAMS,
        out_type=jax.ShapeDtypeStruct((EE, 64), jnp.bfloat16),
        scratch_types=[pltpu.VMEM((GS * CH,), jnp.int32), pltpu.VMEM((GS * CH,), jnp.int32),
                       pltpu.VMEM((GS * CH, 32), jnp.bfloat16),
                       pltpu.VMEM((GS * CH, 32), jnp.bfloat16),
                       pltpu.SemaphoreType.DMA, pltpu.SemaphoreType.DMA],
    )(_ab_sc_body)


def _make_att_sc():
    f32 = jnp.float32
    return functools.partial(
        pl.kernel, mesh=_SC_MESH, compiler_params=_SC_PARAMS,
        out_type=(jax.ShapeDtypeStruct((NC, NP, DD), f32),
                  jax.ShapeDtypeStruct((NC, NP), f32)),
        scratch_types=[pltpu.VMEM((NN,), f32), pltpu.VMEM((NN,), f32),
                       pltpu.VMEM((GS * CH,), jnp.int32), pltpu.VMEM((GS * CH,), jnp.int32),
                       pltpu.VMEM((GS, CH), jnp.int32),
                       pltpu.VMEM((CH,), f32), pltpu.VMEM((2, CH, DD), f32),
                       pltpu.SemaphoreType.DMA,
                       pltpu.VMEM_SHARED((NP, DD), f32), pltpu.VMEM_SHARED((NP,), f32)],
    )(_att_sc_body)


def _make_nk_sc(ew, nch):
    f32 = jnp.float32
    return functools.partial(
        pl.kernel, mesh=_SC_MESH, compiler_params=_SC_PARAMS,
        out_type=jax.ShapeDtypeStruct((NC, NP, DD), f32),
        scratch_types=[pltpu.VMEM((ew,), jnp.int32), pltpu.VMEM((nch, CH), jnp.int32),
                       pltpu.VMEM((2, CH, DD), f32),
                       pltpu.SemaphoreType.DMA,
                       pltpu.VMEM_SHARED((NP, DD), f32)],
    )(_mk_nk_body(ew, nch))


def kernel(x, edge_index, W_pn, b_pn, ln1_g, ln1_b, W_pk, b_pk, ln2_g, ln2_b,
           W_pe, b_pe, W_an, b_an, W_ih, b_ih, W_hh, b_hh, ln3_g, ln3_b,
           W_cp, b_cp, ln4_g, ln4_b):
    f32 = jnp.float32
    wpn32 = jnp.pad(W_pn, ((0, 0), (0, 12)))
    bpn32 = jnp.pad(b_pn, (0, 12)).reshape(1, 32)
    g1 = jnp.pad(ln1_g, (0, 12)).reshape(1, 32)
    b1 = jnp.pad(ln1_b, (0, 12)).reshape(1, 32)
    w640 = jnp.pad(W_pk.reshape(KP, KP, DD), ((0, 0), (0, 12), (0, 0))).reshape(640, DD).astype(jnp.bfloat16)
    wped = W_pe[:DD].reshape(1, DD)
    wpes = W_pe[DD:].reshape(1, DD)
    bpe = b_pe.reshape(1, 1)
    src1 = edge_index[0]
    dst1 = edge_index[1]
    zero_np = jnp.zeros((NP, DD), f32)
    ra = _np.zeros((64, 640), _np.float32)
    rb = _np.zeros((64, 640), _np.float32)
    for _i in range(KP):
        for _k in range(32):
            ra[_i, _i * 32 + _k] = 1.0
            rb[32 + _k, _i * 32 + _k] = 1.0
    ra = jnp.asarray(ra, dtype=jnp.bfloat16)
    rb = jnp.asarray(rb, dtype=jnp.bfloat16)

    full2 = lambda shp: pl.BlockSpec(shp, lambda i: (0, 0))

    proj, hv, ld, ls = pl.pallas_call(
        _pre_body,
        grid=(NN // BN,),
        in_specs=[pl.BlockSpec((BN, DD), lambda i: (i, 0)),
                  full2((DD, 32)), full2((1, 32)), full2((1, 32)), full2((1, 32)),
                  full2((DD, DD)), full2((1, DD)), full2((1, DD)), full2((1, DD)),
                  full2((1, 1))],
        out_specs=[pl.BlockSpec((BN, 32), lambda i: (i, 0)),
                   pl.BlockSpec((BN, DD), lambda i: (i, 0)),
                   pl.BlockSpec((BN, 1), lambda i: (i, 0)),
                   pl.BlockSpec((BN, 1), lambda i: (i, 0))],
        out_shape=[jax.ShapeDtypeStruct((NN, 32), jnp.bfloat16),
                   jax.ShapeDtypeStruct((NN, DD), f32),
                   jax.ShapeDtypeStruct((NN, 1), f32),
                   jax.ShapeDtypeStruct((NN, 1), f32)],
    )(x, wpn32, bpn32, g1, b1, W_an, b_an.reshape(1, DD), wped, wpes, bpe)

    ld1 = ld.reshape(NN)
    ls1 = ls.reshape(NN)

    ab_arr = _make_ab_sc()(proj, src1, dst1)
    num_p, s_p = _make_att_sc()(hv, ld1, ls1, src1, dst1, zero_np)

    EA = 192000
    nblk_a = EA // BE

    def kron_call(nblk, blk_off, row_off):
        return pl.pallas_call(
            _kron_body,
            grid=(nblk,),
            in_specs=[pl.BlockSpec((BE, 64), lambda i: (i + blk_off, 0)),
                      full2((64, 640)), full2((64, 640)),
                      full2((640, DD)), full2((1, DD)), full2((1, DD)), full2((1, DD))],
            out_specs=pl.BlockSpec((BE, DD), lambda i: (i, 0)),
            out_shape=jax.ShapeDtypeStruct((nblk * BE, DD), f32),
        )(ab_arr, ra, rb, w640, b_pk.reshape(1, DD), ln2_g.reshape(1, DD),
          ln2_b.reshape(1, DD))

    ke_a = kron_call(nblk_a, 0, 0)
    nk_pa = _make_nk_sc(EA // NW, EA // NW // CH)(ke_a, dst1[:EA], zero_np)
    ke_b = kron_call(EE // BE - nblk_a, nblk_a, EA)
    nk_pb = _make_nk_sc((EE - EA) // NW, (EE - EA) // NW // CH)(ke_b, dst1[EA:], zero_np)

    out = pl.pallas_call(
        _fin_body,
        grid=(NN // BN,),
        in_specs=[pl.BlockSpec((BN, DD), lambda i: (i, 0)),
                  pl.BlockSpec((BN, DD), lambda i: (i, 0)),
                  pl.BlockSpec((BN, DD), lambda i: (i, 0)),
                  pl.BlockSpec((BN, 1), lambda i: (i, 0)),
                  pl.BlockSpec((BN, 1), lambda i: (i, 0)),
                  pl.BlockSpec((BN, DD), lambda i: (i, 0)),
                  pl.BlockSpec((BN, DD), lambda i: (i, 0)),
                  pl.BlockSpec((BN, DD), lambda i: (i, 0)),
                  pl.BlockSpec((BN, DD), lambda i: (i, 0)),
                  full2((DD, 3 * DD)), full2((1, 3 * DD)),
                  full2((DD, 3 * DD)), full2((1, 3 * DD)),
                  full2((1, DD)), full2((1, DD)),
                  full2((DD, DD)), full2((DD, DD)), full2((1, DD)),
                  full2((1, DD)), full2((1, DD))],
        out_specs=pl.BlockSpec((BN, DD), lambda i: (i, 0)),
        out_shape=jax.ShapeDtypeStruct((NN, DD), f32),
    )(x, num_p[0, :NN], num_p[1, :NN], s_p[0, :NN].reshape(NN, 1), s_p[1, :NN].reshape(NN, 1),
      nk_pa[0, :NN], nk_pa[1, :NN], nk_pb[0, :NN], nk_pb[1, :NN],
      W_ih.T, b_ih.reshape(1, 3 * DD), W_hh.T, b_hh.reshape(1, 3 * DD),
      ln3_g.reshape(1, DD), ln3_b.reshape(1, DD),
      W_cp[:DD], W_cp[DD:], b_cp.reshape(1, DD),
      ln4_g.reshape(1, DD), ln4_b.reshape(1, DD))
    return out
